# baseline (speedup 1.0000x reference)
import dataclasses
import functools

import jax
import jax.numpy as jnp
from jax import lax
from jax.experimental import pallas as pl
from jax.experimental.pallas import tpu as pltpu
from jax.experimental.pallas import tpu_sc as plsc

N = 10000
E = 320000
EMB = 128
HID = 128
B = 64

NC = 2
NS = 16
NW = NC * NS
LANES = 16

NPAD = 10240
ROWS_PER_SUB = NPAD // NS
CH = 64
CHUNKS = 160
BLK = 16
NBLK = CHUNKS // BLK
NBUF = 4
EPW = CH * CHUNKS
EPAD = EPW * NW

XPW = NPAD // NW
XCH = 64
XCHUNKS = XPW // XCH


def _sc_params():
  cp = pltpu.CompilerParams()
  if "needs_layout_passes" in pltpu.CompilerParams.__dataclass_fields__:
    cp = dataclasses.replace(cp, needs_layout_passes=False)
  return cp


@functools.cache
def _vmesh():
  return plsc.VectorSubcoreMesh(
      core_axis_name="c", subcore_axis_name="s", num_cores=NC, num_subcores=NS)


def _emb_deg_body(emb_hbm, idx_hbm, dst_hbm, out_hbm, deg_hbm,
                  idx_v, rows_v, dst_w, deg_v, sem):
  cid = lax.axis_index("c")
  sid = lax.axis_index("s")
  wid = cid * NS + sid
  zeros16 = jnp.zeros((LANES,), jnp.float32)
  ones16 = jnp.ones((LANES,), jnp.float32)

  pltpu.sync_copy(idx_hbm.at[cid, sid], idx_v)

  @pl.loop(0, XCHUNKS)
  def _(j):
    pltpu.async_copy(emb_hbm.at[idx_v.at[j]], rows_v, sem).wait()
    pltpu.sync_copy(rows_v, out_hbm.at[pl.ds(wid * XPW + j * XCH, XCH)])

  @pl.loop(0, NPAD, step=LANES)
  def _(i):
    deg_v[pl.ds(i, LANES)] = zeros16

  pltpu.sync_copy(dst_hbm.at[cid, sid], dst_w)

  @pl.loop(0, CHUNKS)
  def _(j):
    for t in range(CH // LANES):
      idx16 = dst_w[j, pl.ds(t * LANES, LANES)]
      plsc.addupdate_scatter(deg_v, [idx16], ones16)

  pltpu.sync_copy(deg_v, deg_hbm.at[wid])


@jax.jit
def _emb_deg(emb, idx4, dst4):
  k = pl.kernel(
      _emb_deg_body,
      out_type=(jax.ShapeDtypeStruct((NPAD, EMB), jnp.float32),
                jax.ShapeDtypeStruct((NW, NPAD), jnp.float32)),
      mesh=_vmesh(),
      scratch_types=[
          pltpu.VMEM((XCHUNKS, XCH), jnp.int32),
          pltpu.VMEM((XCH, EMB), jnp.float32),
          pltpu.VMEM((CHUNKS, CH), jnp.int32),
          pltpu.VMEM((NPAD,), jnp.float32),
          pltpu.SemaphoreType.DMA,
      ],
      compiler_params=_sc_params(),
  )
  return k(emb, idx4, dst4)


def _edge_agg_body(m_hbm, e_hbm, part_hbm,
                   idx_a, idx_b, bufs, acc, sem_a, sem_b, *sems):
  cid = lax.axis_index("c")
  sid = lax.axis_index("s")
  zeros16 = jnp.zeros((LANES,), jnp.float32)
  buf0 = bufs.at[0]

  @pl.loop(0, CH)
  def _(i):
    for cc in range(EMB // LANES):
      bufs[0, i, pl.ds(cc * LANES, LANES)] = zeros16

  @pl.loop(0, ROWS_PER_SUB, step=CH)
  def _(rr):
    pltpu.sync_copy(buf0, acc.at[pl.ds(sid * ROWS_PER_SUB + rr, CH)])

  plsc.subcore_barrier()

  def fire_idx(b, ibuf, sem):
    pltpu.async_copy(e_hbm.at[cid, sid, pl.ds(b * BLK, BLK)], ibuf, sem)

  def wait_idx(b, ibuf, sem):
    pltpu.make_async_copy(e_hbm.at[cid, sid, pl.ds(b * BLK, BLK)], ibuf,
                          sem).wait()

  def fire(ibuf, j, k):
    pltpu.async_copy(m_hbm.at[ibuf.at[j, 0]], bufs.at[k], sems[k])

  def drain_scatter(ibuf, j, k):
    pltpu.make_async_copy(m_hbm.at[ibuf.at[j, 0]], bufs.at[k],
                          sems[k]).wait()
    pltpu.sync_copy(bufs.at[k], acc.at[ibuf.at[j, 1]], add=True)

  def process_block(ibuf):
    for k in range(NBUF - 1):
      fire(ibuf, k, k)

    @pl.loop(0, BLK // NBUF)
    def _(t):
      for k in range(NBUF):
        j = t * NBUF + k

        @pl.when(j + NBUF - 1 < BLK)
        def _():
          fire(ibuf, j + NBUF - 1, (k + NBUF - 1) % NBUF)

        drain_scatter(ibuf, j, k)

  fire_idx(0, idx_a, sem_a)

  @pl.loop(0, NBLK // 2)
  def _(bb):
    b = bb * 2
    fire_idx(b + 1, idx_b, sem_b)
    wait_idx(b, idx_a, sem_a)
    process_block(idx_a)

    @pl.when(bb < NBLK // 2 - 1)
    def _():
      fire_idx(b + 2, idx_a, sem_a)

    wait_idx(b + 1, idx_b, sem_b)
    process_block(idx_b)

  plsc.subcore_barrier()

  pltpu.sync_copy(
      acc.at[pl.ds(sid * ROWS_PER_SUB, ROWS_PER_SUB)],
      part_hbm.at[cid, pl.ds(sid * ROWS_PER_SUB, ROWS_PER_SUB)])


@jax.jit
def _edge_agg(m, e5):
  k = pl.kernel(
      _edge_agg_body,
      out_type=jax.ShapeDtypeStruct((NC, NPAD, EMB), jnp.float32),
      mesh=_vmesh(),
      scratch_types=[
          pltpu.VMEM((BLK, 2, CH), jnp.int32),
          pltpu.VMEM((BLK, 2, CH), jnp.int32),
          pltpu.VMEM((NBUF, CH, EMB), jnp.float32),
          pltpu.VMEM_SHARED((NPAD, EMB), jnp.float32),
          pltpu.SemaphoreType.DMA,
          pltpu.SemaphoreType.DMA,
      ] + [pltpu.SemaphoreType.DMA] * NBUF,
      compiler_params=_sc_params(),
  )
  return k(m, e5)


def _ln_mm(h, g, b, wl, bl, wr):
  mu = jnp.mean(h, axis=-1, keepdims=True)
  var = jnp.mean((h - mu) ** 2, axis=-1, keepdims=True)
  hl = (h - mu) * lax.rsqrt(var + 1e-5) * g + b
  m = jnp.dot(hl, wl, preferred_element_type=jnp.float32)
  r = jnp.dot(hl, wr, preferred_element_type=jnp.float32) + bl
  return m, r


def _tc_a_body(h_ref, g_ref, b_ref, wl_ref, bl_ref, wr_ref, m_ref, r_ref):
  m, r = _ln_mm(h_ref[...], g_ref[...], b_ref[...], wl_ref[...],
                bl_ref[...], wr_ref[...])
  m_ref[...] = m
  r_ref[...] = r


@jax.jit
def _tc_a(h, g, b, wl, bl, wr):
  return pl.pallas_call(
      _tc_a_body,
      out_shape=(jax.ShapeDtypeStruct((NPAD, HID), jnp.float32),
                 jax.ShapeDtypeStruct((NPAD, HID), jnp.float32)),
  )(h, g, b, wl, bl, wr)


def _tc_b_body(part_ref, degp_ref, r_ref, g_ref, b_ref, wl_ref, bl_ref,
               wr_ref, m_ref, rn_ref, recip_ref):
  deg = jnp.sum(degp_ref[...], axis=0)
  recip = 1.0 / jnp.maximum(deg, 1.0)
  recip_ref[...] = recip[:, None]
  p = part_ref[...]
  agg = (p[0] + p[1]) * recip[:, None]
  h = jax.nn.relu(agg + r_ref[...])
  m, r = _ln_mm(h, g_ref[...], b_ref[...], wl_ref[...], bl_ref[...],
                wr_ref[...])
  m_ref[...] = m
  rn_ref[...] = r


@jax.jit
def _tc_b(part, degp, r, g, b, wl, bl, wr):
  return pl.pallas_call(
      _tc_b_body,
      out_shape=(jax.ShapeDtypeStruct((NPAD, HID), jnp.float32),
                 jax.ShapeDtypeStruct((NPAD, HID), jnp.float32),
                 jax.ShapeDtypeStruct((NPAD, 1), jnp.float32)),
  )(part, degp, r, g, b, wl, bl, wr)


def _tc_c_body(part_ref, recip_ref, r_ref, batch_ref, w1_ref, b1_ref,
               w2_ref, b2_ref, out_ref):
  p = part_ref[...]
  agg = (p[0] + p[1]) * recip_ref[...]
  h = jax.nn.relu(agg + r_ref[...])
  seg = lax.broadcasted_iota(jnp.int32, (B, NPAD), 0)
  onehot = (seg == batch_ref[...]).astype(jnp.float32)
  s = jnp.dot(onehot, h, preferred_element_type=jnp.float32)
  cnt = jnp.sum(onehot, axis=1)
  gm = s / jnp.maximum(cnt, 1.0)[:, None]
  z = jax.nn.relu(jnp.dot(gm, w1_ref[...],
                          preferred_element_type=jnp.float32) + b1_ref[...])
  out_ref[...] = jnp.dot(z, w2_ref[...],
                         preferred_element_type=jnp.float32) + b2_ref[...]


@jax.jit
def _tc_c(part, recip, r, batch2, w1, b1, w2, b2):
  return pl.pallas_call(
      _tc_c_body,
      out_shape=jax.ShapeDtypeStruct((B, 2), jnp.float32),
  )(part, recip, r, batch2, w1, b1, w2, b2)


def kernel(x, edge_index, batch, emb, ln0_g, ln0_b, Wl0, bl0, Wr0,
           ln1_g, ln1_b, Wl1, bl1, Wr1, W1, b1, W2, b2):
  x_pad = jnp.pad(x.astype(jnp.int32), (0, NPAD - N)).reshape(
      NC, NS, XCHUNKS, XCH)
  src = edge_index[0].astype(jnp.int32)
  dst = edge_index[1].astype(jnp.int32)
  pad_e = EPAD - E
  src_pad = jnp.pad(src, (0, pad_e)).reshape(NC, NS, CHUNKS, CH)
  dst_fill = N + jnp.arange(pad_e, dtype=jnp.int32) % (NPAD - N)
  dst_pad = jnp.concatenate([dst, dst_fill]).reshape(NC, NS, CHUNKS, CH)
  e5 = jnp.stack([src_pad, dst_pad], axis=3)
  batch_pad = jnp.pad(batch.astype(jnp.int32), (0, NPAD - N),
                      constant_values=B).reshape(1, NPAD)

  h0, degp = _emb_deg(emb, x_pad, dst_pad)
  m0, r0 = _tc_a(h0, ln0_g, ln0_b, Wl0, bl0, Wr0)
  part0 = _edge_agg(m0, e5)
  m1, r1, recip = _tc_b(part0, degp, r0, ln1_g, ln1_b, Wl1, bl1, Wr1)
  part1 = _edge_agg(m1, e5)
  return _tc_c(part1, recip, r1, batch_pad, W1, b1, W2, b2)


# speedup vs baseline: 1.3135x; 1.3135x over previous
import dataclasses
import functools

import jax
import jax.numpy as jnp
from jax import lax
from jax.experimental import pallas as pl
from jax.experimental.pallas import tpu as pltpu
from jax.experimental.pallas import tpu_sc as plsc

N = 10000
E = 320000
EMB = 128
HID = 128
B = 64

NC = 2
NS = 16
NW = NC * NS
LANES = 16

NPAD = 10240
ROWS_PER_SUB = NPAD // NS
CH = 128
CHUNKS = 80
BLK = 10
NBLK = CHUNKS // BLK
EPW = CH * CHUNKS
EPAD = EPW * NW

XPW = NPAD // NW
XCH = 64
XCHUNKS = XPW // XCH


def _sc_params():
  cp = pltpu.CompilerParams()
  if "needs_layout_passes" in pltpu.CompilerParams.__dataclass_fields__:
    cp = dataclasses.replace(cp, needs_layout_passes=False)
  return cp


@functools.cache
def _vmesh():
  return plsc.VectorSubcoreMesh(
      core_axis_name="c", subcore_axis_name="s", num_cores=NC, num_subcores=NS)


def _emb_deg_body(emb_hbm, idx_hbm, dst_hbm, out_hbm, deg_hbm,
                  idx_v, rows_v, dst_w, deg_v, sem):
  cid = lax.axis_index("c")
  sid = lax.axis_index("s")
  wid = cid * NS + sid
  zeros16 = jnp.zeros((LANES,), jnp.float32)
  ones16 = jnp.ones((LANES,), jnp.float32)

  pltpu.sync_copy(idx_hbm.at[cid, sid], idx_v)

  @pl.loop(0, XCHUNKS)
  def _(j):
    pltpu.async_copy(emb_hbm.at[idx_v.at[j]], rows_v, sem).wait()
    pltpu.sync_copy(rows_v, out_hbm.at[pl.ds(wid * XPW + j * XCH, XCH)])

  @pl.loop(0, NPAD, step=LANES)
  def _(i):
    deg_v[pl.ds(i, LANES)] = zeros16

  pltpu.sync_copy(dst_hbm.at[cid, sid], dst_w)

  @pl.loop(0, CHUNKS)
  def _(j):
    for t in range(CH // LANES):
      idx16 = dst_w[j, pl.ds(t * LANES, LANES)]
      plsc.addupdate_scatter(deg_v, [idx16], ones16)

  pltpu.sync_copy(deg_v, deg_hbm.at[wid])


@jax.jit
def _emb_deg(emb, idx4, dst4):
  k = pl.kernel(
      _emb_deg_body,
      out_type=(jax.ShapeDtypeStruct((NPAD, EMB), jnp.float32),
                jax.ShapeDtypeStruct((NW, NPAD), jnp.float32)),
      mesh=_vmesh(),
      scratch_types=[
          pltpu.VMEM((XCHUNKS, XCH), jnp.int32),
          pltpu.VMEM((XCH, EMB), jnp.float32),
          pltpu.VMEM((CHUNKS, CH), jnp.int32),
          pltpu.VMEM((NPAD,), jnp.float32),
          pltpu.SemaphoreType.DMA,
      ],
      compiler_params=_sc_params(),
  )
  return k(emb, idx4, dst4)


def _edge_agg_body(m_hbm, e_hbm, part_hbm,
                   idx_a, idx_b, buf0, buf1, acc, sem_a, sem_b, sem0, sem1):
  cid = lax.axis_index("c")
  sid = lax.axis_index("s")
  zeros16 = jnp.zeros((LANES,), jnp.float32)

  @pl.loop(0, CH)
  def _(i):
    for cc in range(EMB // LANES):
      buf0[i, pl.ds(cc * LANES, LANES)] = zeros16

  @pl.loop(0, ROWS_PER_SUB, step=CH)
  def _(rr):
    pltpu.sync_copy(buf0, acc.at[pl.ds(sid * ROWS_PER_SUB + rr, CH)])

  plsc.subcore_barrier()

  mc = m_hbm.at[cid]

  def fire_idx(b, ibuf, sem):
    pltpu.async_copy(e_hbm.at[cid, sid, pl.ds(b * BLK, BLK)], ibuf, sem)

  def wait_idx(b, ibuf, sem):
    pltpu.make_async_copy(e_hbm.at[cid, sid, pl.ds(b * BLK, BLK)], ibuf,
                          sem).wait()

  def fire(ibuf, j, buf, sem):
    pltpu.async_copy(mc.at[ibuf.at[j, 0]], buf, sem)

  def drain_scatter(ibuf, j, buf, sem):
    pltpu.make_async_copy(mc.at[ibuf.at[j, 0]], buf, sem).wait()
    pltpu.sync_copy(buf, acc.at[ibuf.at[j, 1]], add=True)

  def process_block(ibuf):
    fire(ibuf, 0, buf0, sem0)

    @pl.loop(0, BLK // 2)
    def _(t):
      j = t * 2
      fire(ibuf, j + 1, buf1, sem1)
      drain_scatter(ibuf, j, buf0, sem0)

      @pl.when(j + 2 < BLK)
      def _():
        fire(ibuf, j + 2, buf0, sem0)

      drain_scatter(ibuf, j + 1, buf1, sem1)

  fire_idx(0, idx_a, sem_a)

  @pl.loop(0, NBLK // 2)
  def _(bb):
    b = bb * 2
    fire_idx(b + 1, idx_b, sem_b)
    wait_idx(b, idx_a, sem_a)
    process_block(idx_a)

    @pl.when(bb < NBLK // 2 - 1)
    def _():
      fire_idx(b + 2, idx_a, sem_a)

    wait_idx(b + 1, idx_b, sem_b)
    process_block(idx_b)

  plsc.subcore_barrier()

  pltpu.sync_copy(
      acc.at[pl.ds(sid * ROWS_PER_SUB, ROWS_PER_SUB)],
      part_hbm.at[cid, pl.ds(sid * ROWS_PER_SUB, ROWS_PER_SUB)])


@jax.jit
def _edge_agg(m2, e5):
  k = pl.kernel(
      _edge_agg_body,
      out_type=jax.ShapeDtypeStruct((NC, NPAD, EMB), jnp.float32),
      mesh=_vmesh(),
      scratch_types=[
          pltpu.VMEM((BLK, 2, CH), jnp.int32),
          pltpu.VMEM((BLK, 2, CH), jnp.int32),
          pltpu.VMEM((CH, EMB), jnp.float32),
          pltpu.VMEM((CH, EMB), jnp.float32),
          pltpu.VMEM_SHARED((NPAD, EMB), jnp.float32),
          pltpu.SemaphoreType.DMA,
          pltpu.SemaphoreType.DMA,
          pltpu.SemaphoreType.DMA,
          pltpu.SemaphoreType.DMA,
      ],
      compiler_params=_sc_params(),
  )
  return k(m2, e5)


def _ln_mm(h, g, b, wl, bl, wr):
  mu = jnp.mean(h, axis=-1, keepdims=True)
  var = jnp.mean((h - mu) ** 2, axis=-1, keepdims=True)
  hl = (h - mu) * lax.rsqrt(var + 1e-5) * g + b
  m = jnp.dot(hl, wl, preferred_element_type=jnp.float32)
  r = jnp.dot(hl, wr, preferred_element_type=jnp.float32) + bl
  m2 = jnp.stack([m, m])
  return m2, r


def _tc_a_body(h_ref, g_ref, b_ref, wl_ref, bl_ref, wr_ref, m_ref, r_ref):
  m, r = _ln_mm(h_ref[...], g_ref[...], b_ref[...], wl_ref[...],
                bl_ref[...], wr_ref[...])
  m_ref[...] = m
  r_ref[...] = r


@jax.jit
def _tc_a(h, g, b, wl, bl, wr):
  return pl.pallas_call(
      _tc_a_body,
      out_shape=(jax.ShapeDtypeStruct((NC, NPAD, HID), jnp.float32),
                 jax.ShapeDtypeStruct((NPAD, HID), jnp.float32)),
  )(h, g, b, wl, bl, wr)


def _tc_b_body(part_ref, degp_ref, r_ref, g_ref, b_ref, wl_ref, bl_ref,
               wr_ref, m_ref, rn_ref, recip_ref):
  deg = jnp.sum(degp_ref[...], axis=0)
  recip = 1.0 / jnp.maximum(deg, 1.0)
  recip_ref[...] = recip[:, None]
  p = part_ref[...]
  agg = (p[0] + p[1]) * recip[:, None]
  h = jax.nn.relu(agg + r_ref[...])
  m, r = _ln_mm(h, g_ref[...], b_ref[...], wl_ref[...], bl_ref[...],
                wr_ref[...])
  m_ref[...] = m
  rn_ref[...] = r


@jax.jit
def _tc_b(part, degp, r, g, b, wl, bl, wr):
  return pl.pallas_call(
      _tc_b_body,
      out_shape=(jax.ShapeDtypeStruct((NC, NPAD, HID), jnp.float32),
                 jax.ShapeDtypeStruct((NPAD, HID), jnp.float32),
                 jax.ShapeDtypeStruct((NPAD, 1), jnp.float32)),
  )(part, degp, r, g, b, wl, bl, wr)


def _tc_c_body(part_ref, recip_ref, r_ref, batch_ref, w1_ref, b1_ref,
               w2_ref, b2_ref, out_ref):
  p = part_ref[...]
  agg = (p[0] + p[1]) * recip_ref[...]
  h = jax.nn.relu(agg + r_ref[...])
  seg = lax.broadcasted_iota(jnp.int32, (B, NPAD), 0)
  onehot = (seg == batch_ref[...]).astype(jnp.float32)
  s = jnp.dot(onehot, h, preferred_element_type=jnp.float32)
  cnt = jnp.sum(onehot, axis=1)
  gm = s / jnp.maximum(cnt, 1.0)[:, None]
  z = jax.nn.relu(jnp.dot(gm, w1_ref[...],
                          preferred_element_type=jnp.float32) + b1_ref[...])
  out_ref[...] = jnp.dot(z, w2_ref[...],
                         preferred_element_type=jnp.float32) + b2_ref[...]


@jax.jit
def _tc_c(part, recip, r, batch2, w1, b1, w2, b2):
  return pl.pallas_call(
      _tc_c_body,
      out_shape=jax.ShapeDtypeStruct((B, 2), jnp.float32),
  )(part, recip, r, batch2, w1, b1, w2, b2)


def kernel(x, edge_index, batch, emb, ln0_g, ln0_b, Wl0, bl0, Wr0,
           ln1_g, ln1_b, Wl1, bl1, Wr1, W1, b1, W2, b2):
  x_pad = jnp.pad(x.astype(jnp.int32), (0, NPAD - N)).reshape(
      NC, NS, XCHUNKS, XCH)
  src = edge_index[0].astype(jnp.int32)
  dst = edge_index[1].astype(jnp.int32)
  pad_e = EPAD - E
  src_pad = jnp.pad(src, (0, pad_e))
  dst_fill = N + jnp.arange(pad_e, dtype=jnp.int32) % (NPAD - N)
  dst_pad = jnp.concatenate([dst, dst_fill])
  dst4 = dst_pad.reshape(NC, NS, CHUNKS, CH)
  e5 = jnp.stack([src_pad.reshape(NC, NS, CHUNKS, CH),
                  dst_pad.reshape(NC, NS, CHUNKS, CH)],
                 axis=3)
  batch_pad = jnp.pad(batch.astype(jnp.int32), (0, NPAD - N),
                      constant_values=B).reshape(1, NPAD)

  h0, degp = _emb_deg(emb, x_pad, dst4)
  m0, r0 = _tc_a(h0, ln0_g, ln0_b, Wl0, bl0, Wr0)
  part0 = _edge_agg(m0, e5)
  m1, r1, recip = _tc_b(part0, degp, r0, ln1_g, ln1_b, Wl1, bl1, Wr1)
  part1 = _edge_agg(m1, e5)
  return _tc_c(part1, recip, r1, batch_pad, W1, b1, W2, b2)


# speedup vs baseline: 1.8126x; 1.3800x over previous
import dataclasses
import functools

import jax
import jax.numpy as jnp
from jax import lax
from jax.experimental import pallas as pl
from jax.experimental.pallas import tpu as pltpu
from jax.experimental.pallas import tpu_sc as plsc

N = 10000
E = 320000
EMB = 128
HID = 128
B = 64

NC = 2
NS = 16
NW = NC * NS
LANES = 16

NPAD = 10240
ROWS_PER_SUB = NPAD // NS
CH = 128
CHUNKS = 80
BLK = 10
EPW = CH * CHUNKS
EPAD = EPW * NW
SCHUNKS = 160
BLK0 = 12
BLK1 = 4

XPW = NPAD // NW
XCH = 64
XCHUNKS = XPW // XCH


def _sc_params():
  cp = pltpu.CompilerParams()
  if "needs_layout_passes" in pltpu.CompilerParams.__dataclass_fields__:
    cp = dataclasses.replace(cp, needs_layout_passes=False)
  return cp


@functools.cache
def _vmesh():
  return plsc.VectorSubcoreMesh(
      core_axis_name="c", subcore_axis_name="s", num_cores=NC, num_subcores=NS)


def _emb_deg_body(emb_hbm, idx_hbm, dst_hbm, out_hbm, deg_hbm,
                  idx_v, rows_v, dst_w, deg_v, sem):
  cid = lax.axis_index("c")
  sid = lax.axis_index("s")
  wid = cid * NS + sid
  zeros16 = jnp.zeros((LANES,), jnp.float32)
  ones16 = jnp.ones((LANES,), jnp.float32)

  pltpu.sync_copy(idx_hbm.at[cid, sid], idx_v)

  @pl.loop(0, XCHUNKS)
  def _(j):
    pltpu.async_copy(emb_hbm.at[idx_v.at[j]], rows_v, sem).wait()
    pltpu.sync_copy(rows_v, out_hbm.at[pl.ds(wid * XPW + j * XCH, XCH)])

  @pl.loop(0, NPAD, step=LANES)
  def _(i):
    deg_v[pl.ds(i, LANES)] = zeros16

  pltpu.sync_copy(dst_hbm.at[cid, sid], dst_w)

  @pl.loop(0, CHUNKS)
  def _(j):
    for t in range(CH // LANES):
      idx16 = dst_w[j, pl.ds(t * LANES, LANES)]
      plsc.addupdate_scatter(deg_v, [idx16], ones16)

  pltpu.sync_copy(deg_v, deg_hbm.at[wid])


@jax.jit
def _emb_deg(emb, idx4, dst4):
  k = pl.kernel(
      _emb_deg_body,
      out_type=(jax.ShapeDtypeStruct((NPAD, EMB), jnp.float32),
                jax.ShapeDtypeStruct((NW, NPAD), jnp.float32)),
      mesh=_vmesh(),
      scratch_types=[
          pltpu.VMEM((XCHUNKS, XCH), jnp.int32),
          pltpu.VMEM((XCH, EMB), jnp.float32),
          pltpu.VMEM((CHUNKS, CH), jnp.int32),
          pltpu.VMEM((NPAD,), jnp.float32),
          pltpu.SemaphoreType.DMA,
      ],
      compiler_params=_sc_params(),
  )
  return k(emb, idx4, dst4)


def _edge_agg_body(m_hbm, e_hbm, part_hbm,
                   idx_a, idx_b, buf0, buf1, acc, sem_a, sem_b, sem0, sem1):
  cid = lax.axis_index("c")
  sid = lax.axis_index("s")
  zeros16 = jnp.zeros((LANES,), jnp.float32)

  @pl.loop(0, CH)
  def _(i):
    for cc in range(EMB // LANES):
      buf0[i, pl.ds(cc * LANES, LANES)] = zeros16

  @pl.loop(0, ROWS_PER_SUB, step=CH)
  def _(rr):
    pltpu.sync_copy(buf0, acc.at[pl.ds(sid * ROWS_PER_SUB + rr, CH)])

  plsc.subcore_barrier()

  mc = m_hbm.at[cid]

  def fire_idx(b, ibuf, sem):
    pltpu.async_copy(e_hbm.at[sid, pl.ds(b * BLK, BLK)], ibuf, sem)

  def wait_idx(b, ibuf, sem):
    pltpu.make_async_copy(e_hbm.at[sid, pl.ds(b * BLK, BLK)], ibuf,
                          sem).wait()

  def fire(ibuf, j, buf, sem):
    pltpu.async_copy(mc.at[ibuf.at[j, 0]], buf, sem)

  def drain_scatter(ibuf, j, buf, sem):
    pltpu.make_async_copy(mc.at[ibuf.at[j, 0]], buf, sem).wait()
    pltpu.sync_copy(buf, acc.at[ibuf.at[j, 1]], add=True)

  def process_block(ibuf):
    fire(ibuf, 0, buf0, sem0)

    @pl.loop(0, BLK // 2)
    def _(t):
      j = t * 2
      fire(ibuf, j + 1, buf1, sem1)
      drain_scatter(ibuf, j, buf0, sem0)

      @pl.when(j + 2 < BLK)
      def _():
        fire(ibuf, j + 2, buf0, sem0)

      drain_scatter(ibuf, j + 1, buf1, sem1)

  def run_pipeline(base, nblk):
    fire_idx(base, idx_a, sem_a)

    @pl.loop(0, nblk // 2)
    def _(bb):
      b = base + bb * 2
      fire_idx(b + 1, idx_b, sem_b)
      wait_idx(b, idx_a, sem_a)
      process_block(idx_a)

      @pl.when(bb < nblk // 2 - 1)
      def _():
        fire_idx(b + 2, idx_a, sem_a)

      wait_idx(b + 1, idx_b, sem_b)
      process_block(idx_b)

  @pl.when(cid == 0)
  def _():
    run_pipeline(0, BLK0)

  @pl.when(cid == 1)
  def _():
    run_pipeline(BLK0, BLK1)

  plsc.subcore_barrier()

  pltpu.sync_copy(
      acc.at[pl.ds(sid * ROWS_PER_SUB, ROWS_PER_SUB)],
      part_hbm.at[cid, pl.ds(sid * ROWS_PER_SUB, ROWS_PER_SUB)])


@jax.jit
def _edge_agg(m2, e5):
  k = pl.kernel(
      _edge_agg_body,
      out_type=jax.ShapeDtypeStruct((NC, NPAD, EMB), jnp.float32),
      mesh=_vmesh(),
      scratch_types=[
          pltpu.VMEM((BLK, 2, CH), jnp.int32),
          pltpu.VMEM((BLK, 2, CH), jnp.int32),
          pltpu.VMEM((CH, EMB), jnp.float32),
          pltpu.VMEM((CH, EMB), jnp.float32),
          pltpu.VMEM_SHARED((NPAD, EMB), jnp.float32),
          pltpu.SemaphoreType.DMA,
          pltpu.SemaphoreType.DMA,
          pltpu.SemaphoreType.DMA,
          pltpu.SemaphoreType.DMA,
      ],
      compiler_params=_sc_params(),
  )
  return k(m2, e5)


def _ln_mm(h, g, b, wl, bl, wr):
  mu = jnp.mean(h, axis=-1, keepdims=True)
  var = jnp.mean((h - mu) ** 2, axis=-1, keepdims=True)
  hl = (h - mu) * lax.rsqrt(var + 1e-5) * g + b
  m = jnp.dot(hl, wl, preferred_element_type=jnp.float32)
  r = jnp.dot(hl, wr, preferred_element_type=jnp.float32) + bl
  m2 = jnp.stack([m, m])
  return m2, r


def _tc_a_body(h_ref, g_ref, b_ref, wl_ref, bl_ref, wr_ref, m_ref, r_ref):
  m, r = _ln_mm(h_ref[...], g_ref[...], b_ref[...], wl_ref[...],
                bl_ref[...], wr_ref[...])
  m_ref[...] = m
  r_ref[...] = r


@jax.jit
def _tc_a(h, g, b, wl, bl, wr):
  return pl.pallas_call(
      _tc_a_body,
      out_shape=(jax.ShapeDtypeStruct((NC, NPAD, HID), jnp.float32),
                 jax.ShapeDtypeStruct((NPAD, HID), jnp.float32)),
  )(h, g, b, wl, bl, wr)


def _tc_b_body(part_ref, degp_ref, r_ref, g_ref, b_ref, wl_ref, bl_ref,
               wr_ref, m_ref, rn_ref, recip_ref):
  deg = jnp.sum(degp_ref[...], axis=0)
  recip = 1.0 / jnp.maximum(deg, 1.0)
  recip_ref[...] = recip[:, None]
  p = part_ref[...]
  agg = (p[0] + p[1]) * recip[:, None]
  h = jax.nn.relu(agg + r_ref[...])
  m, r = _ln_mm(h, g_ref[...], b_ref[...], wl_ref[...], bl_ref[...],
                wr_ref[...])
  m_ref[...] = m
  rn_ref[...] = r


@jax.jit
def _tc_b(part, degp, r, g, b, wl, bl, wr):
  return pl.pallas_call(
      _tc_b_body,
      out_shape=(jax.ShapeDtypeStruct((NC, NPAD, HID), jnp.float32),
                 jax.ShapeDtypeStruct((NPAD, HID), jnp.float32),
                 jax.ShapeDtypeStruct((NPAD, 1), jnp.float32)),
  )(part, degp, r, g, b, wl, bl, wr)


def _tc_c_body(part_ref, recip_ref, r_ref, batch_ref, w1_ref, b1_ref,
               w2_ref, b2_ref, out_ref):
  p = part_ref[...]
  agg = (p[0] + p[1]) * recip_ref[...]
  h = jax.nn.relu(agg + r_ref[...])
  seg = lax.broadcasted_iota(jnp.int32, (B, NPAD), 0)
  onehot = (seg == batch_ref[...]).astype(jnp.float32)
  s = jnp.dot(onehot, h, preferred_element_type=jnp.float32)
  cnt = jnp.sum(onehot, axis=1)
  gm = s / jnp.maximum(cnt, 1.0)[:, None]
  z = jax.nn.relu(jnp.dot(gm, w1_ref[...],
                          preferred_element_type=jnp.float32) + b1_ref[...])
  out_ref[...] = jnp.dot(z, w2_ref[...],
                         preferred_element_type=jnp.float32) + b2_ref[...]


@jax.jit
def _tc_c(part, recip, r, batch2, w1, b1, w2, b2):
  return pl.pallas_call(
      _tc_c_body,
      out_shape=jax.ShapeDtypeStruct((B, 2), jnp.float32),
  )(part, recip, r, batch2, w1, b1, w2, b2)


def kernel(x, edge_index, batch, emb, ln0_g, ln0_b, Wl0, bl0, Wr0,
           ln1_g, ln1_b, Wl1, bl1, Wr1, W1, b1, W2, b2):
  x_pad = jnp.pad(x.astype(jnp.int32), (0, NPAD - N)).reshape(
      NC, NS, XCHUNKS, XCH)
  src = edge_index[0].astype(jnp.int32)
  dst = edge_index[1].astype(jnp.int32)
  pad_e = EPAD - E
  src_pad = jnp.pad(src, (0, pad_e))
  dst_fill = N + jnp.arange(pad_e, dtype=jnp.int32) % (NPAD - N)
  dst_pad = jnp.concatenate([dst, dst_fill])
  dst4 = dst_pad.reshape(NC, NS, CHUNKS, CH)
  e5 = jnp.stack([src_pad.reshape(NS, SCHUNKS, CH),
                  dst_pad.reshape(NS, SCHUNKS, CH)],
                 axis=2)
  batch_pad = jnp.pad(batch.astype(jnp.int32), (0, NPAD - N),
                      constant_values=B).reshape(1, NPAD)

  h0, degp = _emb_deg(emb, x_pad, dst4)
  m0, r0 = _tc_a(h0, ln0_g, ln0_b, Wl0, bl0, Wr0)
  part0 = _edge_agg(m0, e5)
  m1, r1, recip = _tc_b(part0, degp, r0, ln1_g, ln1_b, Wl1, bl1, Wr1)
  part1 = _edge_agg(m1, e5)
  return _tc_c(part1, recip, r1, batch_pad, W1, b1, W2, b2)
